# baseline (speedup 1.0000x reference)
import functools

import jax
import jax.numpy as jnp
from jax import lax
from jax.experimental import pallas as pl
from jax.experimental.pallas import tpu as pltpu
from jax.experimental.pallas import tpu_sc as plsc

N = 10000
M = 32
D = 128
DN = 16
NM = N * M
NW = 32
CH = 128
NCH = 80
EPW = NCH * CH
NM_PAD = NW * EPW

AT = 200
ET = AT * M
GRID = N // AT
EPS = 1e-5


GRP = 1
GR = GRP * CH
NG = NCH // GRP


def _sc_gather(x, idx2d):
    mesh = plsc.VectorSubcoreMesh(core_axis_name="c", subcore_axis_name="s")

    @functools.partial(
        pl.kernel,
        out_type=jax.ShapeDtypeStruct((NM_PAD, D), jnp.float32),
        scratch_types=[
            pltpu.VMEM((NCH, CH), jnp.int32),
            pltpu.VMEM((GR, D), jnp.float32),
            pltpu.VMEM((GR, D), jnp.float32),
            pltpu.VMEM_SHARED((N, D), jnp.float32),
        ]
        + [pltpu.SemaphoreType.DMA] * (2 * GRP + 2),
        mesh=mesh,
    )
    def k(x_hbm, idx_hbm, out_hbm, idx_v, rows0, rows1, table_s, *sems):
        rows = (rows0, rows1)
        gsem = (sems[:GRP], sems[GRP:2 * GRP])
        ssem = sems[2 * GRP:]
        sid = lax.axis_index("s")
        wid = lax.axis_index("c") * 16 + sid

        @pl.when(sid == 0)
        def _stage():
            pltpu.sync_copy(x_hbm, table_s)

        pltpu.sync_copy(idx_hbm.at[pl.ds(wid * NCH, NCH)], idx_v)
        plsc.subcore_barrier()

        def fire_gathers(t, r):
            for c in range(GRP):
                pltpu.async_copy(
                    table_s.at[idx_v.at[t * GRP + c]],
                    rows[r].at[pl.ds(c * CH, CH)],
                    gsem[r][c],
                )

        def drain_gathers(t, r):
            for c in range(GRP):
                pltpu.make_async_copy(
                    table_s.at[idx_v.at[t * GRP + c]],
                    rows[r].at[pl.ds(c * CH, CH)],
                    gsem[r][c],
                ).wait()

        def fire_store(t, r):
            pltpu.async_copy(
                rows[r], out_hbm.at[pl.ds(wid * EPW + t * GR, GR)], ssem[r]
            )

        def wait_store(r):
            pltpu.make_async_copy(
                rows[r], out_hbm.at[pl.ds(0, GR)], ssem[r]
            ).wait()

        fire_gathers(0, 0)
        drain_gathers(0, 0)
        fire_store(0, 0)
        fire_gathers(1, 1)

        def body(u, carry):
            for kk in range(2):
                t = 1 + u * 2 + kk
                r = (1 + kk) % 2
                drain_gathers(t, r)
                fire_store(t, r)
                wait_store(1 - r)
                fire_gathers(t + 1, 1 - r)
            return carry

        lax.fori_loop(0, (NG - 2) // 2, body, 0)
        drain_gathers(NG - 1, 1)
        fire_store(NG - 1, 1)
        wait_store(0)
        wait_store(1)

    return k(x, idx2d)


def _embed_body(a_ref, w_ref, b_ref, o_ref):
    o_ref[...] = (
        jnp.dot(a_ref[...], w_ref[...], preferred_element_type=jnp.float32)
        + b_ref[...]
    )


def _embed(atom, w_t, b):
    return pl.pallas_call(
        _embed_body,
        out_shape=jax.ShapeDtypeStruct((N, D), jnp.float32),
    )(atom, w_t, b)


def _gated_chunk(s, g_ref, nbr_ref, wn_ref, we_ref):
    nb = nbr_ref[...].reshape(ET, DN)
    gated = (
        jnp.dot(g_ref[...], wn_ref[...], preferred_element_type=jnp.float32)
        + jnp.dot(nb, we_ref[...], preferred_element_type=jnp.float32)
    )
    s_rep = jnp.broadcast_to(s[:, None, :], (AT, M, 2 * D)).reshape(ET, 2 * D)
    return gated + s_rep


def _stats_body(s_ref, g_ref, nbr_ref, wn_ref, we_ref,
                sum_ref, ssq_ref):
    i = pl.program_id(0)

    @pl.when(i == 0)
    def _init():
        sum_ref[...] = jnp.zeros_like(sum_ref)
        ssq_ref[...] = jnp.zeros_like(ssq_ref)

    gated = _gated_chunk(s_ref[...], g_ref, nbr_ref, wn_ref, we_ref)
    sum_ref[...] += jnp.sum(gated, axis=0)[None, :]
    ssq_ref[...] += jnp.sum(gated * gated, axis=0)[None, :]


def _apply_body(s_ref, g_ref, nbr_ref, wn_ref, we_ref, s1_ref, q1_ref,
                g1_ref, b1_ref, ns_ref, sum2_ref, ssq2_ref):
    i = pl.program_id(0)

    @pl.when(i == 0)
    def _init():
        sum2_ref[...] = jnp.zeros_like(sum2_ref)
        ssq2_ref[...] = jnp.zeros_like(ssq2_ref)

    mu = s1_ref[...] / NM
    var = q1_ref[...] / NM - mu * mu
    alpha = g1_ref[...] * lax.rsqrt(var + EPS)
    beta = b1_ref[...] - mu * alpha

    gated = _gated_chunk(s_ref[...], g_ref, nbr_ref, wn_ref, we_ref)
    gn = gated * alpha + beta
    filt = jax.nn.sigmoid(gn[:, :D])
    core = jax.nn.softplus(gn[:, D:])
    prod = (filt * core).reshape(AT, M, D)
    ns = jnp.sum(prod, axis=1)
    ns_ref[...] = ns
    sum2_ref[...] += jnp.sum(ns, axis=0)[None, :]
    ssq2_ref[...] += jnp.sum(ns * ns, axis=0)[None, :]


def _prep_s_body(x_ref, ws_ref, b_ref, s_ref):
    s_ref[...] = (
        jnp.dot(x_ref[...], ws_ref[...], preferred_element_type=jnp.float32)
        + b_ref[...]
    )


def _conv_passes(x, g, nbr, ws_t, wn_t, we_t, b, g1, b1):
    stat_shape = jax.ShapeDtypeStruct((1, 2 * D), jnp.float32)
    stat_spec = pl.BlockSpec((1, 2 * D), lambda i: (0, 0))
    data_specs = [
        pl.BlockSpec((AT, 2 * D), lambda i: (i, 0)),
        pl.BlockSpec((ET, D), lambda i: (i, 0)),
        pl.BlockSpec((AT, M, DN), lambda i: (i, 0, 0)),
    ]
    w_specs = [
        pl.BlockSpec((D, 2 * D), lambda i: (0, 0)),
        pl.BlockSpec((DN, 2 * D), lambda i: (0, 0)),
    ]

    s = pl.pallas_call(
        _prep_s_body,
        out_shape=jax.ShapeDtypeStruct((N, 2 * D), jnp.float32),
    )(x, ws_t, b)

    s1, q1 = pl.pallas_call(
        _stats_body,
        grid=(GRID,),
        in_specs=data_specs + w_specs,
        out_specs=[stat_spec, stat_spec],
        out_shape=[stat_shape, stat_shape],
    )(s, g, nbr, wn_t, we_t)

    ns, s2, q2 = pl.pallas_call(
        _apply_body,
        grid=(GRID,),
        in_specs=data_specs + w_specs + [stat_spec, stat_spec,
                                         stat_spec, stat_spec],
        out_specs=[
            pl.BlockSpec((AT, D), lambda i: (i, 0)),
            pl.BlockSpec((1, D), lambda i: (0, 0)),
            pl.BlockSpec((1, D), lambda i: (0, 0)),
        ],
        out_shape=[
            jax.ShapeDtypeStruct((N, D), jnp.float32),
            jax.ShapeDtypeStruct((1, D), jnp.float32),
            jax.ShapeDtypeStruct((1, D), jnp.float32),
        ],
    )(s, g, nbr, wn_t, we_t, s1, q1, g1, b1)
    return ns, s2, q2


def _resid_body(x_ref, ns_ref, s2_ref, q2_ref, g2_ref, b2_ref, o_ref):
    mu = s2_ref[...] / N
    var = q2_ref[...] / N - mu * mu
    alpha = g2_ref[...] * lax.rsqrt(var + EPS)
    beta = b2_ref[...] - mu * alpha
    o_ref[...] = jax.nn.softplus(x_ref[...] + ns_ref[...] * alpha + beta)


def _resid(x, ns, s2, q2, g2, b2):
    return pl.pallas_call(
        _resid_body,
        out_shape=jax.ShapeDtypeStruct((N, D), jnp.float32),
    )(x, ns, s2, q2, g2, b2)


def _head_body(x_ref, wf_ref, bf_ref, w1_ref, b1_ref, wo_ref, bo_ref, o_ref):
    pooled = jnp.sum(x_ref[...].reshape(100, 100, D), axis=1) * (1.0 / 100.0)
    crys_fea = (
        jnp.dot(pooled, wf_ref[...], preferred_element_type=jnp.float32)
        + bf_ref[...]
    )
    fused = jax.nn.relu(
        jnp.dot(crys_fea, w1_ref[...], preferred_element_type=jnp.float32)
        + b1_ref[...]
    )
    o_ref[...] = (
        jnp.dot(fused, wo_ref[...], preferred_element_type=jnp.float32)
        + bo_ref[...]
    )


def _head(x, wf_t, bf, w1_t, b1, wo_t, bo):
    return pl.pallas_call(
        _head_body,
        out_shape=jax.ShapeDtypeStruct((100, 1), jnp.float32),
    )(x, wf_t, bf, w1_t, b1, wo_t, bo)


def kernel(atom, nbr, idx, crys, mono_target1, mono_target2, params):
    del crys, mono_target1, mono_target2
    emb = params["embedding"]
    x = _embed(atom, emb["W"].T, emb["b"][None, :])

    idx_pad = jnp.concatenate(
        [idx.reshape(-1), jnp.zeros((NM_PAD - NM,), jnp.int32)]
    ).reshape(NM_PAD // CH, CH)

    for p in params["convs"]:
        w = p["fc"]["W"]
        ws_t = w[:, :D].T
        wn_t = w[:, D:2 * D].T
        we_t = w[:, 2 * D:].T
        b = p["fc"]["b"][None, :]
        g = _sc_gather(x, idx_pad)
        ns, s2, q2 = _conv_passes(
            x, g, nbr, ws_t, wn_t, we_t, b,
            p["bn1_g"][None, :], p["bn1_b"][None, :],
        )
        x = _resid(x, ns, s2, q2, p["bn2_g"][None, :], p["bn2_b"][None, :])

    fc = params["fc"]
    fu = params["fusion_fc1"]
    fo = params["fc_out"]
    return _head(
        x,
        fc["W"].T, fc["b"][None, :],
        fu["W"].T, fu["b"][None, :],
        fo["W"].T, fo["b"][None, :],
    )


# speedup vs baseline: 1.0571x; 1.0571x over previous
import functools

import jax
import jax.numpy as jnp
from jax import lax
from jax.experimental import pallas as pl
from jax.experimental.pallas import tpu as pltpu
from jax.experimental.pallas import tpu_sc as plsc

N = 10000
M = 32
D = 128
DN = 16
NM = N * M
NW = 32
CH = 128
NCH = 80
EPW = NCH * CH
NM_PAD = NW * EPW

AT = 200
ET = AT * M
GRID = N // AT
EPS = 1e-5


GRP = 1
GR = GRP * CH
NG = NCH // GRP


def _sc_gather(x, idx2d):
    mesh = plsc.VectorSubcoreMesh(core_axis_name="c", subcore_axis_name="s")

    @functools.partial(
        pl.kernel,
        out_type=jax.ShapeDtypeStruct((NM_PAD, D), jnp.float32),
        scratch_types=[
            pltpu.VMEM((NCH, CH), jnp.int32),
            pltpu.VMEM((GR, D), jnp.float32),
            pltpu.VMEM((GR, D), jnp.float32),
            pltpu.VMEM_SHARED((N, D), jnp.float32),
        ]
        + [pltpu.SemaphoreType.DMA] * (2 * GRP + 2),
        mesh=mesh,
    )
    def k(x_hbm, idx_hbm, out_hbm, idx_v, rows0, rows1, table_s, *sems):
        rows = (rows0, rows1)
        gsem = (sems[:GRP], sems[GRP:2 * GRP])
        ssem = sems[2 * GRP:]
        sid = lax.axis_index("s")
        wid = lax.axis_index("c") * 16 + sid

        @pl.when(sid == 0)
        def _stage():
            pltpu.sync_copy(x_hbm, table_s)

        pltpu.sync_copy(idx_hbm.at[pl.ds(wid * NCH, NCH)], idx_v)
        plsc.subcore_barrier()

        def fire_gathers(t, r):
            for c in range(GRP):
                pltpu.async_copy(
                    table_s.at[idx_v.at[t * GRP + c]],
                    rows[r].at[pl.ds(c * CH, CH)],
                    gsem[r][c],
                )

        def drain_gathers(t, r):
            for c in range(GRP):
                pltpu.make_async_copy(
                    table_s.at[idx_v.at[t * GRP + c]],
                    rows[r].at[pl.ds(c * CH, CH)],
                    gsem[r][c],
                ).wait()

        def fire_store(t, r):
            pltpu.async_copy(
                rows[r], out_hbm.at[pl.ds(wid * EPW + t * GR, GR)], ssem[r]
            )

        def wait_store(r):
            pltpu.make_async_copy(
                rows[r], out_hbm.at[pl.ds(0, GR)], ssem[r]
            ).wait()

        fire_gathers(0, 0)
        drain_gathers(0, 0)
        fire_store(0, 0)
        fire_gathers(1, 1)

        def body(u, carry):
            for kk in range(2):
                t = 1 + u * 2 + kk
                r = (1 + kk) % 2
                drain_gathers(t, r)
                fire_store(t, r)
                wait_store(1 - r)
                fire_gathers(t + 1, 1 - r)
            return carry

        lax.fori_loop(0, (NG - 2) // 2, body, 0)
        drain_gathers(NG - 1, 1)
        fire_store(NG - 1, 1)
        wait_store(0)
        wait_store(1)

    return k(x, idx2d)


def _embed_body(a_ref, w_ref, b_ref, o_ref):
    o_ref[...] = (
        jnp.dot(a_ref[...], w_ref[...], preferred_element_type=jnp.float32)
        + b_ref[...]
    )


def _embed(atom, w_t, b):
    return pl.pallas_call(
        _embed_body,
        out_shape=jax.ShapeDtypeStruct((N, D), jnp.float32),
    )(atom, w_t, b)


def _gated_chunk(x_ref, g_ref, nbr_ref, ws_ref, wn_ref, we_ref, b_ref):
    s = (
        jnp.dot(x_ref[...], ws_ref[...], preferred_element_type=jnp.float32)
        + b_ref[...]
    )
    nb = nbr_ref[...].reshape(ET, DN)
    gated = (
        jnp.dot(g_ref[...], wn_ref[...], preferred_element_type=jnp.float32)
        + jnp.dot(nb, we_ref[...], preferred_element_type=jnp.float32)
    )
    s_rep = jnp.broadcast_to(s[:, None, :], (AT, M, 2 * D)).reshape(ET, 2 * D)
    return gated + s_rep


def _stats_body(x_ref, g_ref, nbr_ref, ws_ref, wn_ref, we_ref, b_ref,
                sum_ref, ssq_ref):
    i = pl.program_id(0)

    @pl.when(i == 0)
    def _init():
        sum_ref[...] = jnp.zeros_like(sum_ref)
        ssq_ref[...] = jnp.zeros_like(ssq_ref)

    gated = _gated_chunk(x_ref, g_ref, nbr_ref, ws_ref, wn_ref, we_ref, b_ref)
    sum_ref[...] += jnp.sum(gated, axis=0)[None, :]
    ssq_ref[...] += jnp.sum(gated * gated, axis=0)[None, :]


def _apply_body(x_ref, g_ref, nbr_ref, ws_ref, wn_ref, we_ref, b_ref,
                sum_ref, ssq_ref, g1_ref, b1_ref,
                ns_ref, sum2_ref, ssq2_ref):
    i = pl.program_id(0)

    @pl.when(i == 0)
    def _init():
        sum2_ref[...] = jnp.zeros_like(sum2_ref)
        ssq2_ref[...] = jnp.zeros_like(ssq2_ref)

    mu = sum_ref[...] / NM
    var = ssq_ref[...] / NM - mu * mu
    alpha = g1_ref[...] * lax.rsqrt(var + EPS)
    beta = b1_ref[...] - mu * alpha

    gated = _gated_chunk(x_ref, g_ref, nbr_ref, ws_ref, wn_ref, we_ref, b_ref)
    gn = gated * alpha + beta
    filt = jax.nn.sigmoid(gn[:, :D])
    core = jax.nn.softplus(gn[:, D:])
    prod = (filt * core).reshape(AT, M, D)
    ns = jnp.sum(prod, axis=1)
    ns_ref[...] = ns
    sum2_ref[...] += jnp.sum(ns, axis=0)[None, :]
    ssq2_ref[...] += jnp.sum(ns * ns, axis=0)[None, :]


def _conv_passes(x, g, nbr, ws_t, wn_t, we_t, b, g1, b1):
    stat_shape = jax.ShapeDtypeStruct((1, 2 * D), jnp.float32)
    w_specs = [
        pl.BlockSpec((D, 2 * D), lambda i: (0, 0)),
        pl.BlockSpec((D, 2 * D), lambda i: (0, 0)),
        pl.BlockSpec((DN, 2 * D), lambda i: (0, 0)),
        pl.BlockSpec((1, 2 * D), lambda i: (0, 0)),
    ]
    data_specs = [
        pl.BlockSpec((AT, D), lambda i: (i, 0)),
        pl.BlockSpec((ET, D), lambda i: (i, 0)),
        pl.BlockSpec((AT, M, DN), lambda i: (i, 0, 0)),
    ]
    stat_spec = pl.BlockSpec((1, 2 * D), lambda i: (0, 0))

    s1, q1 = pl.pallas_call(
        _stats_body,
        grid=(GRID,),
        in_specs=data_specs + w_specs,
        out_specs=[stat_spec, stat_spec],
        out_shape=[stat_shape, stat_shape],
    )(x, g, nbr, ws_t, wn_t, we_t, b)

    ns, s2, q2 = pl.pallas_call(
        _apply_body,
        grid=(GRID,),
        in_specs=data_specs + w_specs + [
            stat_spec,
            stat_spec,
            pl.BlockSpec((1, 2 * D), lambda i: (0, 0)),
            pl.BlockSpec((1, 2 * D), lambda i: (0, 0)),
        ],
        out_specs=[
            pl.BlockSpec((AT, D), lambda i: (i, 0)),
            pl.BlockSpec((1, D), lambda i: (0, 0)),
            pl.BlockSpec((1, D), lambda i: (0, 0)),
        ],
        out_shape=[
            jax.ShapeDtypeStruct((N, D), jnp.float32),
            jax.ShapeDtypeStruct((1, D), jnp.float32),
            jax.ShapeDtypeStruct((1, D), jnp.float32),
        ],
    )(x, g, nbr, ws_t, wn_t, we_t, b, s1, q1, g1, b1)
    return ns, s2, q2


def _resid_body(x_ref, ns_ref, s2_ref, q2_ref, g2_ref, b2_ref, o_ref):
    mu = s2_ref[...] / N
    var = q2_ref[...] / N - mu * mu
    alpha = g2_ref[...] * lax.rsqrt(var + EPS)
    beta = b2_ref[...] - mu * alpha
    o_ref[...] = jax.nn.softplus(x_ref[...] + ns_ref[...] * alpha + beta)


def _resid(x, ns, s2, q2, g2, b2):
    return pl.pallas_call(
        _resid_body,
        out_shape=jax.ShapeDtypeStruct((N, D), jnp.float32),
    )(x, ns, s2, q2, g2, b2)


def _head_body(x_ref, wf_ref, bf_ref, w1_ref, b1_ref, wo_ref, bo_ref, o_ref):
    pooled = jnp.sum(x_ref[...].reshape(100, 100, D), axis=1) * (1.0 / 100.0)
    crys_fea = (
        jnp.dot(pooled, wf_ref[...], preferred_element_type=jnp.float32)
        + bf_ref[...]
    )
    fused = jax.nn.relu(
        jnp.dot(crys_fea, w1_ref[...], preferred_element_type=jnp.float32)
        + b1_ref[...]
    )
    o_ref[...] = (
        jnp.dot(fused, wo_ref[...], preferred_element_type=jnp.float32)
        + bo_ref[...]
    )


def _head(x, wf_t, bf, w1_t, b1, wo_t, bo):
    return pl.pallas_call(
        _head_body,
        out_shape=jax.ShapeDtypeStruct((100, 1), jnp.float32),
    )(x, wf_t, bf, w1_t, b1, wo_t, bo)


def kernel(atom, nbr, idx, crys, mono_target1, mono_target2, params):
    del crys, mono_target1, mono_target2
    emb = params["embedding"]
    x = _embed(atom, emb["W"].T, emb["b"][None, :])

    idx_pad = jnp.concatenate(
        [idx.reshape(-1), jnp.zeros((NM_PAD - NM,), jnp.int32)]
    ).reshape(NM_PAD // CH, CH)

    for p in params["convs"]:
        w = p["fc"]["W"]
        ws_t = w[:, :D].T
        wn_t = w[:, D:2 * D].T
        we_t = w[:, 2 * D:].T
        b = p["fc"]["b"][None, :]
        g = _sc_gather(x, idx_pad)
        ns, s2, q2 = _conv_passes(
            x, g, nbr, ws_t, wn_t, we_t, b,
            p["bn1_g"][None, :], p["bn1_b"][None, :],
        )
        x = _resid(x, ns, s2, q2, p["bn2_g"][None, :], p["bn2_b"][None, :])

    fc = params["fc"]
    fu = params["fusion_fc1"]
    fo = params["fc_out"]
    return _head(
        x,
        fc["W"].T, fc["b"][None, :],
        fu["W"].T, fu["b"][None, :],
        fo["W"].T, fo["b"][None, :],
    )


# speedup vs baseline: 1.0984x; 1.0391x over previous
import functools

import jax
import jax.numpy as jnp
from jax import lax
from jax.experimental import pallas as pl
from jax.experimental.pallas import tpu as pltpu
from jax.experimental.pallas import tpu_sc as plsc

N = 10000
M = 32
D = 128
DN = 16
NM = N * M
NW = 32
CH = 128
NCH = 80
EPW = NCH * CH
NM_PAD = NW * EPW

AT = 400
ET = AT * M
GRID = N // AT
EPS = 1e-5


GRP = 1
GR = GRP * CH
NG = NCH // GRP


def _sc_gather(x, idx2d):
    mesh = plsc.VectorSubcoreMesh(core_axis_name="c", subcore_axis_name="s")

    @functools.partial(
        pl.kernel,
        out_type=jax.ShapeDtypeStruct((NM_PAD, D), jnp.float32),
        scratch_types=[
            pltpu.VMEM((NCH, CH), jnp.int32),
            pltpu.VMEM((GR, D), jnp.float32),
            pltpu.VMEM((GR, D), jnp.float32),
            pltpu.VMEM_SHARED((N, D), jnp.float32),
        ]
        + [pltpu.SemaphoreType.DMA] * (2 * GRP + 2),
        mesh=mesh,
    )
    def k(x_hbm, idx_hbm, out_hbm, idx_v, rows0, rows1, table_s, *sems):
        rows = (rows0, rows1)
        gsem = (sems[:GRP], sems[GRP:2 * GRP])
        ssem = sems[2 * GRP:]
        sid = lax.axis_index("s")
        wid = lax.axis_index("c") * 16 + sid

        @pl.when(sid == 0)
        def _stage():
            pltpu.sync_copy(x_hbm, table_s)

        pltpu.sync_copy(idx_hbm.at[pl.ds(wid * NCH, NCH)], idx_v)
        plsc.subcore_barrier()

        def fire_gathers(t, r):
            for c in range(GRP):
                pltpu.async_copy(
                    table_s.at[idx_v.at[t * GRP + c]],
                    rows[r].at[pl.ds(c * CH, CH)],
                    gsem[r][c],
                )

        def drain_gathers(t, r):
            for c in range(GRP):
                pltpu.make_async_copy(
                    table_s.at[idx_v.at[t * GRP + c]],
                    rows[r].at[pl.ds(c * CH, CH)],
                    gsem[r][c],
                ).wait()

        def fire_store(t, r):
            pltpu.async_copy(
                rows[r], out_hbm.at[pl.ds(wid * EPW + t * GR, GR)], ssem[r]
            )

        def wait_store(r):
            pltpu.make_async_copy(
                rows[r], out_hbm.at[pl.ds(0, GR)], ssem[r]
            ).wait()

        fire_gathers(0, 0)
        drain_gathers(0, 0)
        fire_store(0, 0)
        fire_gathers(1, 1)

        def body(u, carry):
            for kk in range(2):
                t = 1 + u * 2 + kk
                r = (1 + kk) % 2
                drain_gathers(t, r)
                fire_store(t, r)
                wait_store(1 - r)
                fire_gathers(t + 1, 1 - r)
            return carry

        lax.fori_loop(0, (NG - 2) // 2, body, 0)
        drain_gathers(NG - 1, 1)
        fire_store(NG - 1, 1)
        wait_store(0)
        wait_store(1)

    return k(x, idx2d)


def _embed_body(a_ref, w_ref, b_ref, o_ref):
    o_ref[...] = (
        jnp.dot(a_ref[...], w_ref[...], preferred_element_type=jnp.float32)
        + b_ref[...]
    )


def _embed(atom, w_t, b):
    return pl.pallas_call(
        _embed_body,
        out_shape=jax.ShapeDtypeStruct((N, D), jnp.float32),
    )(atom, w_t, b)


def _gated_chunk(x_ref, g_ref, nbr_ref, ws_ref, wn_ref, we_ref, b_ref):
    s = (
        jnp.dot(x_ref[...], ws_ref[...], preferred_element_type=jnp.float32)
        + b_ref[...]
    )
    nb = nbr_ref[...].reshape(ET, DN)
    gated = (
        jnp.dot(g_ref[...], wn_ref[...], preferred_element_type=jnp.float32)
        + jnp.dot(nb, we_ref[...], preferred_element_type=jnp.float32)
    )
    s_rep = jnp.broadcast_to(s[:, None, :], (AT, M, 2 * D)).reshape(ET, 2 * D)
    return gated + s_rep


def _stats_body(x_ref, g_ref, nbr_ref, ws_ref, wn_ref, we_ref, b_ref,
                sum_ref, ssq_ref):
    i = pl.program_id(0)

    @pl.when(i == 0)
    def _init():
        sum_ref[...] = jnp.zeros_like(sum_ref)
        ssq_ref[...] = jnp.zeros_like(ssq_ref)

    gated = _gated_chunk(x_ref, g_ref, nbr_ref, ws_ref, wn_ref, we_ref, b_ref)
    sum_ref[...] += jnp.sum(gated, axis=0)[None, :]
    ssq_ref[...] += jnp.sum(gated * gated, axis=0)[None, :]


def _apply_body(x_ref, g_ref, nbr_ref, ws_ref, wn_ref, we_ref, b_ref,
                sum_ref, ssq_ref, g1_ref, b1_ref,
                ns_ref, sum2_ref, ssq2_ref):
    i = pl.program_id(0)

    @pl.when(i == 0)
    def _init():
        sum2_ref[...] = jnp.zeros_like(sum2_ref)
        ssq2_ref[...] = jnp.zeros_like(ssq2_ref)

    mu = sum_ref[...] / NM
    var = ssq_ref[...] / NM - mu * mu
    alpha = g1_ref[...] * lax.rsqrt(var + EPS)
    beta = b1_ref[...] - mu * alpha

    gated = _gated_chunk(x_ref, g_ref, nbr_ref, ws_ref, wn_ref, we_ref, b_ref)
    gn = gated * alpha + beta
    filt = jax.nn.sigmoid(gn[:, :D])
    core = jax.nn.softplus(gn[:, D:])
    prod = (filt * core).reshape(AT, M, D)
    ns = jnp.sum(prod, axis=1)
    ns_ref[...] = ns
    sum2_ref[...] += jnp.sum(ns, axis=0)[None, :]
    ssq2_ref[...] += jnp.sum(ns * ns, axis=0)[None, :]


def _conv_passes(x, g, nbr, ws_t, wn_t, we_t, b, g1, b1):
    stat_shape = jax.ShapeDtypeStruct((1, 2 * D), jnp.float32)
    w_specs = [
        pl.BlockSpec((D, 2 * D), lambda i: (0, 0)),
        pl.BlockSpec((D, 2 * D), lambda i: (0, 0)),
        pl.BlockSpec((DN, 2 * D), lambda i: (0, 0)),
        pl.BlockSpec((1, 2 * D), lambda i: (0, 0)),
    ]
    data_specs = [
        pl.BlockSpec((AT, D), lambda i: (i, 0)),
        pl.BlockSpec((ET, D), lambda i: (i, 0)),
        pl.BlockSpec((AT, M, DN), lambda i: (i, 0, 0)),
    ]
    stat_spec = pl.BlockSpec((1, 2 * D), lambda i: (0, 0))

    s1, q1 = pl.pallas_call(
        _stats_body,
        grid=(GRID,),
        in_specs=data_specs + w_specs,
        out_specs=[stat_spec, stat_spec],
        out_shape=[stat_shape, stat_shape],
    )(x, g, nbr, ws_t, wn_t, we_t, b)

    ns, s2, q2 = pl.pallas_call(
        _apply_body,
        grid=(GRID,),
        in_specs=data_specs + w_specs + [
            stat_spec,
            stat_spec,
            pl.BlockSpec((1, 2 * D), lambda i: (0, 0)),
            pl.BlockSpec((1, 2 * D), lambda i: (0, 0)),
        ],
        out_specs=[
            pl.BlockSpec((AT, D), lambda i: (i, 0)),
            pl.BlockSpec((1, D), lambda i: (0, 0)),
            pl.BlockSpec((1, D), lambda i: (0, 0)),
        ],
        out_shape=[
            jax.ShapeDtypeStruct((N, D), jnp.float32),
            jax.ShapeDtypeStruct((1, D), jnp.float32),
            jax.ShapeDtypeStruct((1, D), jnp.float32),
        ],
    )(x, g, nbr, ws_t, wn_t, we_t, b, s1, q1, g1, b1)
    return ns, s2, q2


def _resid_body(x_ref, ns_ref, s2_ref, q2_ref, g2_ref, b2_ref, o_ref):
    mu = s2_ref[...] / N
    var = q2_ref[...] / N - mu * mu
    alpha = g2_ref[...] * lax.rsqrt(var + EPS)
    beta = b2_ref[...] - mu * alpha
    o_ref[...] = jax.nn.softplus(x_ref[...] + ns_ref[...] * alpha + beta)


def _resid(x, ns, s2, q2, g2, b2):
    return pl.pallas_call(
        _resid_body,
        out_shape=jax.ShapeDtypeStruct((N, D), jnp.float32),
    )(x, ns, s2, q2, g2, b2)


def _head_body(x_ref, wf_ref, bf_ref, w1_ref, b1_ref, wo_ref, bo_ref, o_ref):
    pooled = jnp.sum(x_ref[...].reshape(100, 100, D), axis=1) * (1.0 / 100.0)
    crys_fea = (
        jnp.dot(pooled, wf_ref[...], preferred_element_type=jnp.float32)
        + bf_ref[...]
    )
    fused = jax.nn.relu(
        jnp.dot(crys_fea, w1_ref[...], preferred_element_type=jnp.float32)
        + b1_ref[...]
    )
    o_ref[...] = (
        jnp.dot(fused, wo_ref[...], preferred_element_type=jnp.float32)
        + bo_ref[...]
    )


def _head(x, wf_t, bf, w1_t, b1, wo_t, bo):
    return pl.pallas_call(
        _head_body,
        out_shape=jax.ShapeDtypeStruct((100, 1), jnp.float32),
    )(x, wf_t, bf, w1_t, b1, wo_t, bo)


def kernel(atom, nbr, idx, crys, mono_target1, mono_target2, params):
    del crys, mono_target1, mono_target2
    emb = params["embedding"]
    x = _embed(atom, emb["W"].T, emb["b"][None, :])

    idx_pad = jnp.concatenate(
        [idx.reshape(-1), jnp.zeros((NM_PAD - NM,), jnp.int32)]
    ).reshape(NM_PAD // CH, CH)

    for p in params["convs"]:
        w = p["fc"]["W"]
        ws_t = w[:, :D].T
        wn_t = w[:, D:2 * D].T
        we_t = w[:, 2 * D:].T
        b = p["fc"]["b"][None, :]
        g = _sc_gather(x, idx_pad)
        ns, s2, q2 = _conv_passes(
            x, g, nbr, ws_t, wn_t, we_t, b,
            p["bn1_g"][None, :], p["bn1_b"][None, :],
        )
        x = _resid(x, ns, s2, q2, p["bn2_g"][None, :], p["bn2_b"][None, :])

    fc = params["fc"]
    fu = params["fusion_fc1"]
    fo = params["fc_out"]
    return _head(
        x,
        fc["W"].T, fc["b"][None, :],
        fu["W"].T, fu["b"][None, :],
        fo["W"].T, fo["b"][None, :],
    )
